# Initial kernel scaffold; baseline (speedup 1.0000x reference)
import jax
import jax.numpy as jnp
from jax.experimental import pallas as pl


def kernel(x, W1, b1, w2, b2):
    raise NotImplementedError("write your pallas kernel here")


# speedup vs baseline: 1.2786x; 1.2786x over previous
import jax
import jax.numpy as jnp
from jax.experimental import pallas as pl

_INT_MIN = -2147483648
_INT_MAX = 2147483647


def _scores_kernel(x_ref, w1_ref, b1_ref, w2_ref, out_ref):
    x = x_ref[...].astype(jnp.bfloat16)
    h = jnp.tanh(
        jax.lax.dot_general(x, w1_ref[...].astype(jnp.bfloat16),
                            (((1,), (1,)), ((), ())),
                            preferred_element_type=jnp.float32)
        + b1_ref[...])
    hb = h.astype(jnp.bfloat16).astype(jnp.float32)
    v2 = w2_ref[...].astype(jnp.bfloat16).astype(jnp.float32)
    s = jnp.sum(hb * v2, axis=1)
    out_ref[...] = s[:, None]


def _weights_kernel(s_ref, w_ref):
    s = s_ref[...]
    B, T = s.shape
    n_mask = T // 2
    bits = jax.lax.bitcast_convert_type(s, jnp.int32)
    key = jnp.where(bits >= 0, bits, jnp.int32(_INT_MIN) - bits)

    def vbody(_, lohi):
        lo, hi = lohi
        mid = (lo >> 1) + (hi >> 1) + (lo & hi & 1)
        c = jnp.sum((key <= mid).astype(jnp.int32), axis=1, keepdims=True)
        ge = c >= n_mask
        return (jnp.where(ge, lo, mid + 1), jnp.where(ge, mid, hi))

    lo0 = jnp.full((B, 1), _INT_MIN, jnp.int32)
    hi0 = jnp.full((B, 1), _INT_MAX, jnp.int32)
    K, _ = jax.lax.fori_loop(0, 32, vbody, (lo0, hi0))

    cnt_lt = jnp.sum((key < K).astype(jnp.int32), axis=1, keepdims=True)
    n_eq = n_mask - cnt_lt
    iota = jax.lax.broadcasted_iota(jnp.int32, (B, T), 1)
    eq = key == K

    def ibody(_, lohi):
        lo, hi = lohi
        mid = (lo + hi) >> 1
        c = jnp.sum((eq & (iota <= mid)).astype(jnp.int32), axis=1,
                    keepdims=True)
        ge = c >= n_eq
        return (jnp.where(ge, lo, mid + 1), jnp.where(ge, mid, hi))

    lo0i = jnp.zeros((B, 1), jnp.int32)
    hi0i = jnp.full((B, 1), T - 1, jnp.int32)
    I, _ = jax.lax.fori_loop(0, 13, ibody, (lo0i, hi0i))

    kept = jnp.logical_not((key < K) | (eq & (iota <= I)))
    m = jnp.max(s, axis=1, keepdims=True)
    e = jnp.where(kept, jnp.exp(s - m), 0.0)
    denom = jnp.sum(e, axis=1, keepdims=True)
    w_ref[...] = e / denom


def _scale_kernel(x_ref, w_ref, o_ref):
    o_ref[...] = x_ref[...] * w_ref[...]


def kernel(x, W1, b1, w2, b2):
    B, T, D = x.shape
    del b2
    xf = x.reshape(B * T, D)

    BT = 512
    scores = pl.pallas_call(
        _scores_kernel,
        grid=(B * T // BT,),
        in_specs=[
            pl.BlockSpec((BT, D), lambda i: (i, 0)),
            pl.BlockSpec((D, D), lambda i: (0, 0)),
            pl.BlockSpec((1, D), lambda i: (0, 0)),
            pl.BlockSpec((1, D), lambda i: (0, 0)),
        ],
        out_specs=pl.BlockSpec((BT, 1), lambda i: (i, 0)),
        out_shape=jax.ShapeDtypeStruct((B * T, 1), jnp.float32),
    )(xf, W1, b1.reshape(1, D), w2.reshape(1, D))

    weights = pl.pallas_call(
        _weights_kernel,
        out_shape=jax.ShapeDtypeStruct((B, T), jnp.float32),
    )(scores.reshape(B, T))

    R = 1024
    out = pl.pallas_call(
        _scale_kernel,
        grid=(B * T // R,),
        in_specs=[
            pl.BlockSpec((R, D), lambda i: (i, 0)),
            pl.BlockSpec((R, 1), lambda i: (i, 0)),
        ],
        out_specs=pl.BlockSpec((R, D), lambda i: (i, 0)),
        out_shape=jax.ShapeDtypeStruct((B * T, D), jnp.float32),
    )(xf, weights.reshape(B * T, 1))

    return out.reshape(B, T, D), weights
